# baseline (speedup 1.0000x reference)
import functools

import jax
import jax.numpy as jnp
from jax import lax
from jax.experimental import pallas as pl
from jax.experimental.pallas import tpu as pltpu
from jax.experimental.pallas import tpu_sc as plsc

B = 16384
D = 64
L = 16
NC = 2
NS = 16
NW = NC * NS
BPW = B // NW
CH = 128
NCH = BPW // CH

_mesh = plsc.VectorSubcoreMesh(core_axis_name="c", subcore_axis_name="s")


@functools.partial(
    pl.kernel,
    mesh=_mesh,
    compiler_params=pltpu.CompilerParams(
        needs_layout_passes=False, use_tc_tiling_on_sc=False),
    out_type=jax.ShapeDtypeStruct((B,), jnp.float32),
    scratch_types=[
        pltpu.VMEM((NCH, CH), jnp.int32),
        pltpu.VMEM((NCH, CH), jnp.int32),
        pltpu.VMEM((BPW, D), jnp.float32),
        pltpu.VMEM((BPW, D), jnp.float32),
        pltpu.VMEM((BPW,), jnp.float32),
        pltpu.SemaphoreType.DMA,
    ],
)
def _mf_sc(user_hbm, item_hbm, uf_hbm, if_hbm, out_hbm,
           uidx, iidx, urows, irows, oacc, sem):
    wid = lax.axis_index("s") * NC + lax.axis_index("c")
    base = wid * BPW

    for j in range(NCH):
        pltpu.sync_copy(user_hbm.at[pl.ds(base + j * CH, CH)], uidx.at[j])
        pltpu.sync_copy(item_hbm.at[pl.ds(base + j * CH, CH)], iidx.at[j])

    copies = []
    for j in range(NCH):
        copies.append(
            pltpu.async_copy(uf_hbm.at[uidx.at[j]], urows.at[pl.ds(j * CH, CH)], sem))
        copies.append(
            pltpu.async_copy(if_hbm.at[iidx.at[j]], irows.at[pl.ds(j * CH, CH)], sem))
    for c in copies:
        c.wait()

    lane = lax.broadcasted_iota(jnp.int32, (L,), 0)

    def group_body(g, carry):
        acc = jnp.zeros((L,), jnp.float32)
        for k in range(L):
            b = g * L + k
            p = jnp.zeros((L,), jnp.float32)
            for f in range(0, D, L):
                u = urows[b, pl.ds(f, L)]
                v = irows[b, pl.ds(f, L)]
                p = p + u * v
            acc = jnp.where(lane == k, jnp.sum(p), acc)
        oacc[pl.ds(g * L, L)] = acc
        return carry

    lax.fori_loop(0, BPW // L, group_body, 0)

    pltpu.sync_copy(oacc, out_hbm.at[pl.ds(base, BPW)])


def kernel(user, item, user_factors, item_factors):
    return _mf_sc(user.astype(jnp.int32), item.astype(jnp.int32),
                  user_factors, item_factors)


# speedup vs baseline: 1.1193x; 1.1193x over previous
import functools

import jax
import jax.numpy as jnp
from jax import lax
from jax.experimental import pallas as pl
from jax.experimental.pallas import tpu as pltpu
from jax.experimental.pallas import tpu_sc as plsc

B = 16384
D = 64
L = 16
NC = 2
NS = 16
NW = NC * NS
BPW = B // NW
CH = 128
NSB = BPW // CH

_mesh = plsc.VectorSubcoreMesh(core_axis_name="c", subcore_axis_name="s")


@functools.partial(
    pl.kernel,
    mesh=_mesh,
    compiler_params=pltpu.CompilerParams(needs_layout_passes=False),
    out_type=jax.ShapeDtypeStruct((B,), jnp.float32),
    scratch_types=[
        pltpu.VMEM((NSB, CH), jnp.int32),
        pltpu.VMEM((NSB, CH), jnp.int32),
        pltpu.VMEM((CH, 2 * D), jnp.float32),
        pltpu.VMEM((CH, 2 * D), jnp.float32),
        pltpu.VMEM((CH, 2 * D), jnp.float32),
        pltpu.VMEM((CH, 2 * D), jnp.float32),
        pltpu.VMEM((BPW,), jnp.float32),
        pltpu.SemaphoreType.DMA,
        pltpu.SemaphoreType.DMA,
    ],
)
def _mf_sc(user_hbm, item_hbm, tab_hbm, out_hbm,
           uidx, iidx, ub0, ub1, ib0, ib1, oacc, sem0, sem1):
    wid = lax.axis_index("s") * NC + lax.axis_index("c")
    base = wid * BPW
    ubufs, ibufs, sems = (ub0, ub1), (ib0, ib1), (sem0, sem1)

    for j in range(NSB):
        pltpu.sync_copy(user_hbm.at[pl.ds(base + j * CH, CH)], uidx.at[j])
        pltpu.sync_copy(item_hbm.at[pl.ds(base + j * CH, CH)], iidx.at[j])

    def fire(s):
        p = s % 2
        return (
            pltpu.async_copy(tab_hbm.at[uidx.at[s]], ubufs[p], sems[p]),
            pltpu.async_copy(tab_hbm.at[iidx.at[s]], ibufs[p], sems[p]),
        )

    lane = lax.broadcasted_iota(jnp.int32, (L,), 0)

    inflight = fire(0)
    for s in range(NSB):
        for c in inflight:
            c.wait()
        if s + 1 < NSB:
            nxt = fire(s + 1)
        ub, ib = ubufs[s % 2], ibufs[s % 2]

        def group_body(g, carry, ub=ub, ib=ib, s=s):
            acc = jnp.zeros((L,), jnp.float32)
            for k in range(L):
                b = g * L + k
                p = jnp.zeros((L,), jnp.float32)
                for f in range(0, D, L):
                    u = ub[b, pl.ds(f, L)]
                    v = ib[b, pl.ds(D + f, L)]
                    p = p + u * v
                acc = jnp.where(lane == k, jnp.sum(p), acc)
            oacc[pl.ds(s * CH + g * L, L)] = acc
            return carry

        lax.fori_loop(0, CH // L, group_body, 0)
        if s + 1 < NSB:
            inflight = nxt

    pltpu.sync_copy(oacc, out_hbm.at[pl.ds(base, BPW)])


def kernel(user, item, user_factors, item_factors):
    table = jnp.concatenate([user_factors, item_factors], axis=1)
    return _mf_sc(user.astype(jnp.int32), item.astype(jnp.int32), table)
